# baseline (speedup 1.0000x reference)
import functools

import jax
import jax.numpy as jnp
from jax import lax
from jax.experimental import pallas as pl
from jax.experimental.pallas import tpu as pltpu
from jax.experimental.pallas import tpu_sc as plsc

EMB_DIM = 128
NUM_WORKERS = 32
SEQ_PAD = 56
NBUF = 8


def _make_gather(batch: int, seq: int):
    b_per_w = batch // NUM_WORKERS
    n_groups = b_per_w // NBUF

    mesh = plsc.VectorSubcoreMesh(core_axis_name="c", subcore_axis_name="s")

    @functools.partial(
        pl.kernel,
        mesh=mesh,
        out_type=jax.ShapeDtypeStruct((batch * SEQ_PAD, EMB_DIM), jnp.float32),
        scratch_types=(
            [pltpu.VMEM((b_per_w, SEQ_PAD), jnp.int32)]
            + [pltpu.VMEM((SEQ_PAD, EMB_DIM), jnp.float32) for _ in range(NBUF)]
            + [pltpu.SemaphoreType.DMA for _ in range(2 * NBUF)]
        ),
    )
    def gather_kernel(table_hbm, idx_hbm, out_hbm, idx_v, *scratch):
        bufs = scratch[:NBUF]
        sems_g = scratch[NBUF:2 * NBUF]
        sems_s = scratch[2 * NBUF:]

        wid = lax.axis_index("s") * 2 + lax.axis_index("c")
        pltpu.sync_copy(idx_hbm.at[wid], idx_v)
        base = wid * b_per_w

        def start_gather(j, b):
            pltpu.async_copy(table_hbm.at[idx_v.at[j]], bufs[b], sems_g[b])

        def wait_gather(j, b):
            pltpu.make_async_copy(
                table_hbm.at[idx_v.at[j]], bufs[b], sems_g[b]
            ).wait()

        def start_store(j, b):
            pltpu.async_copy(
                bufs[b],
                out_hbm.at[pl.ds((base + j) * SEQ_PAD, SEQ_PAD)],
                sems_s[b],
            )

        def wait_store(j, b):
            pltpu.make_async_copy(
                bufs[b],
                out_hbm.at[pl.ds((base + j) * SEQ_PAD, SEQ_PAD)],
                sems_s[b],
            ).wait()

        def body(g, carry):
            j0 = g * NBUF
            for b in range(NBUF):
                @pl.when(g > 0)
                def _(b=b):
                    wait_store(j0 + b - NBUF, b)

                start_gather(j0 + b, b)
            for b in range(NBUF):
                wait_gather(j0 + b, b)
                start_store(j0 + b, b)
            return carry

        lax.fori_loop(0, n_groups, body, 0)
        for b in range(NBUF):
            wait_store(b_per_w - NBUF + b, b)

    return gather_kernel


def kernel(inputs, table):
    batch, seq = inputs.shape
    idx = jnp.zeros((batch, SEQ_PAD), jnp.int32)
    idx = idx.at[:, :seq].set(inputs.astype(jnp.int32))
    idx = idx.reshape(NUM_WORKERS, batch // NUM_WORKERS, SEQ_PAD)
    out = _make_gather(batch, seq)(table, idx)
    return out.reshape(batch, SEQ_PAD, EMB_DIM)[:, :seq]


# speedup vs baseline: 6.4943x; 6.4943x over previous
import functools

import jax
import jax.numpy as jnp
from jax import lax
from jax.experimental import pallas as pl
from jax.experimental.pallas import tpu as pltpu
from jax.experimental.pallas import tpu_sc as plsc

EMB_DIM = 128
NUM_WORKERS = 32
SEQ_PAD = 56
NBUF = 8


def _make_gather(batch: int, seq: int):
    b_per_w = batch // NUM_WORKERS
    n_groups = b_per_w // NBUF

    mesh = plsc.VectorSubcoreMesh(core_axis_name="c", subcore_axis_name="s")

    @functools.partial(
        pl.kernel,
        mesh=mesh,
        out_type=jax.ShapeDtypeStruct((batch * SEQ_PAD, EMB_DIM), jnp.float32),
        scratch_types=(
            [pltpu.VMEM((b_per_w, SEQ_PAD), jnp.int32)]
            + [pltpu.VMEM((SEQ_PAD, EMB_DIM), jnp.float32) for _ in range(NBUF)]
            + [pltpu.SemaphoreType.DMA for _ in range(2 * NBUF)]
        ),
    )
    def gather_kernel(table_hbm, idx_hbm, out_hbm, idx_v, *scratch):
        bufs = scratch[:NBUF]
        sems_g = scratch[NBUF:2 * NBUF]
        sems_s = scratch[2 * NBUF:]

        wid = lax.axis_index("s") * 2 + lax.axis_index("c")
        pltpu.sync_copy(idx_hbm.at[wid], idx_v)
        base = wid * b_per_w

        def start_gather(j, b):
            pltpu.async_copy(table_hbm.at[idx_v.at[j]], bufs[b], sems_g[b])

        def wait_gather(j, b):
            pltpu.make_async_copy(
                table_hbm.at[idx_v.at[j]], bufs[b], sems_g[b]
            ).wait()

        def start_store(j, b):
            pltpu.async_copy(
                bufs[b],
                out_hbm.at[pl.ds((base + j) * SEQ_PAD, SEQ_PAD)],
                sems_s[b],
            )

        def wait_store(j, b):
            pltpu.make_async_copy(
                bufs[b],
                out_hbm.at[pl.ds((base + j) * SEQ_PAD, SEQ_PAD)],
                sems_s[b],
            ).wait()

        def body(g, carry):
            j0 = g * NBUF
            for b in range(NBUF):
                @pl.when(g > 0)
                def _(b=b):
                    wait_store(j0 + b - NBUF, b)

                start_gather(j0 + b, b)
            for b in range(NBUF):
                wait_gather(j0 + b, b)
                start_store(j0 + b, b)
            return carry

        lax.fori_loop(0, n_groups, body, 0)
        for b in range(NBUF):
            wait_store(b_per_w - NBUF + b, b)

    return gather_kernel


def kernel(inputs, table):
    batch, seq = inputs.shape
    inputs = inputs.astype(jnp.int32)
    idx = jnp.concatenate([inputs, inputs[:, : SEQ_PAD - seq]], axis=1)
    idx = idx.reshape(NUM_WORKERS, batch // NUM_WORKERS, SEQ_PAD)
    out = _make_gather(batch, seq)(table, idx)
    return out.reshape(batch, SEQ_PAD, EMB_DIM)[:, :seq]


# speedup vs baseline: 7.4994x; 1.1548x over previous
import functools

import jax
import jax.numpy as jnp
from jax import lax
from jax.experimental import pallas as pl
from jax.experimental.pallas import tpu as pltpu
from jax.experimental.pallas import tpu_sc as plsc

EMB_DIM = 128
NUM_WORKERS = 32
SEQ_PAD = 56
NBUF = 8


def _make_gather(batch: int, seq: int):
    b_per_w = batch // NUM_WORKERS
    n_groups = b_per_w // NBUF

    mesh = plsc.VectorSubcoreMesh(core_axis_name="c", subcore_axis_name="s")

    @functools.partial(
        pl.kernel,
        mesh=mesh,
        out_type=jax.ShapeDtypeStruct((batch, seq, EMB_DIM), jnp.float32),
        scratch_types=(
            [pltpu.VMEM((b_per_w, SEQ_PAD), jnp.int32)]
            + [pltpu.VMEM((SEQ_PAD, EMB_DIM), jnp.float32) for _ in range(NBUF)]
            + [pltpu.SemaphoreType.DMA for _ in range(2 * NBUF)]
        ),
    )
    def gather_kernel(table_hbm, idx_hbm, out_hbm, idx_v, *scratch):
        bufs = scratch[:NBUF]
        sems_g = scratch[NBUF:2 * NBUF]
        sems_s = scratch[2 * NBUF:]

        wid = lax.axis_index("s") * 2 + lax.axis_index("c")
        pltpu.sync_copy(idx_hbm.at[wid], idx_v)
        base = wid * b_per_w

        def start_gather(j, b):
            pltpu.async_copy(table_hbm.at[idx_v.at[j]], bufs[b], sems_g[b])

        def wait_gather(j, b):
            pltpu.make_async_copy(
                table_hbm.at[idx_v.at[j]], bufs[b], sems_g[b]
            ).wait()

        def start_store(j, b):
            pltpu.async_copy(
                bufs[b].at[pl.ds(0, seq)], out_hbm.at[base + j], sems_s[b]
            )

        def wait_store(j, b):
            pltpu.make_async_copy(
                bufs[b].at[pl.ds(0, seq)], out_hbm.at[base + j], sems_s[b]
            ).wait()

        def body(g, carry):
            j0 = g * NBUF
            for b in range(NBUF):
                @pl.when(g > 0)
                def _(b=b):
                    wait_store(j0 + b - NBUF, b)

                start_gather(j0 + b, b)
            for b in range(NBUF):
                wait_gather(j0 + b, b)
                start_store(j0 + b, b)
            return carry

        lax.fori_loop(0, n_groups, body, 0)
        for b in range(NBUF):
            wait_store(b_per_w - NBUF + b, b)

    return gather_kernel


def kernel(inputs, table):
    batch, seq = inputs.shape
    inputs = inputs.astype(jnp.int32)
    idx = jnp.concatenate([inputs, inputs[:, : SEQ_PAD - seq]], axis=1)
    idx = idx.reshape(NUM_WORKERS, batch // NUM_WORKERS, SEQ_PAD)
    return _make_gather(batch, seq)(table, idx)


# speedup vs baseline: 13.4962x; 1.7996x over previous
import functools

import jax
import jax.numpy as jnp
from jax import lax
from jax.experimental import pallas as pl
from jax.experimental.pallas import tpu as pltpu
from jax.experimental.pallas import tpu_sc as plsc

EMB_DIM = 128
NUM_WORKERS = 32
CHUNK = 64
NBUF = 10


def _make_gather(n_rows: int):
    n_per_w = n_rows // NUM_WORKERS
    n_chunks = n_per_w // CHUNK
    n_groups = n_chunks // NBUF

    mesh = plsc.VectorSubcoreMesh(core_axis_name="c", subcore_axis_name="s")

    @functools.partial(
        pl.kernel,
        mesh=mesh,
        out_type=jax.ShapeDtypeStruct((n_rows, EMB_DIM), jnp.float32),
        scratch_types=(
            [pltpu.VMEM((n_chunks, CHUNK), jnp.int32)]
            + [pltpu.VMEM((CHUNK, EMB_DIM), jnp.float32) for _ in range(NBUF)]
            + [pltpu.SemaphoreType.DMA for _ in range(2 * NBUF)]
        ),
    )
    def gather_kernel(table_hbm, idx_hbm, out_hbm, idx_v, *scratch):
        bufs = scratch[:NBUF]
        sems_g = scratch[NBUF:2 * NBUF]
        sems_s = scratch[2 * NBUF:]

        wid = lax.axis_index("s") * 2 + lax.axis_index("c")
        pltpu.sync_copy(idx_hbm.at[wid], idx_v)
        base = wid * n_per_w

        def start_gather(j, b):
            pltpu.async_copy(table_hbm.at[idx_v.at[j]], bufs[b], sems_g[b])

        def wait_gather(j, b):
            pltpu.make_async_copy(
                table_hbm.at[idx_v.at[j]], bufs[b], sems_g[b]
            ).wait()

        def start_store(j, b):
            pltpu.async_copy(
                bufs[b], out_hbm.at[pl.ds(base + j * CHUNK, CHUNK)], sems_s[b]
            )

        def wait_store(j, b):
            pltpu.make_async_copy(
                bufs[b], out_hbm.at[pl.ds(base + j * CHUNK, CHUNK)], sems_s[b]
            ).wait()

        def body(g, carry):
            j0 = g * NBUF
            for b in range(NBUF):
                @pl.when(g > 0)
                def _(b=b):
                    wait_store(j0 + b - NBUF, b)

                start_gather(j0 + b, b)
            for b in range(NBUF):
                wait_gather(j0 + b, b)
                start_store(j0 + b, b)
            return carry

        lax.fori_loop(0, n_groups, body, 0)
        for b in range(NBUF):
            wait_store(n_chunks - NBUF + b, b)

    return gather_kernel


def kernel(inputs, table):
    batch, seq = inputs.shape
    n_rows = batch * seq
    idx = inputs.astype(jnp.int32).T.reshape(
        NUM_WORKERS, n_rows // (NUM_WORKERS * CHUNK), CHUNK
    )
    out = _make_gather(n_rows)(table, idx)
    return out.reshape(seq, batch, EMB_DIM).transpose(1, 0, 2)
